# baseline (speedup 1.0000x reference)
import dataclasses

import jax
import jax.numpy as jnp
from jax import lax
from jax.experimental import pallas as pl
from jax.experimental.pallas import tpu as pltpu
from jax.experimental.pallas import tpu_sc as plsc

N = 100000
D = 128
H = 128
B = 1024

NC = 2
NS = 16
NW = NC * NS
SEG_PER_W = B // NW
LANES = 16
DG = D // LANES
CHUNK = 256
BND_WIN = 48
SUB = 128
N_PAD = 100352
PAD_ID = 2047
NSUB = N_PAD // SUB
NSUB_PAD = 896
SLAB_STEP = 48
SLAB_ROWS = 64
ITERS_A = 10
ITERS_B = 7


def _sread(ref, i):
    base = (i // LANES) * LANES
    win = ref[pl.ds(base, LANES)]
    lane = lax.iota(jnp.int32, LANES)
    return jnp.sum(jnp.where(lane == (i - base), win, 0))


def _pool_body(x_hbm, batch_hbm, batch2_hbm, sum_hbm, max_hbm,
               bnd_v, slab_v, stage_v, sub_v, sub_sh, rowi_v, win_v,
               xbuf0, xbuf1, outs_v, outm_v, sem0, sem1):
    sid = lax.axis_index("s")
    wid = lax.axis_index("c") * NS + sid
    seg0 = wid * SEG_PER_W

    lane = lax.iota(jnp.int32, LANES)
    QG = BND_WIN // LANES
    tgts = [seg0 + q * LANES + lane for q in range(QG)]

    scope_bounds = jax.named_scope("sc_stage")
    scope_bounds.__enter__()
    pltpu.sync_copy(batch2_hbm.at[pl.ds(sid * SLAB_STEP, SLAB_ROWS)], slab_v)
    for g in range(4):
        ridx = g * LANES + lane
        stage_v[pl.ds(g * LANES, LANES)] = plsc.load_gather(
            slab_v, [ridx, jnp.zeros((LANES,), jnp.int32)])
    pltpu.sync_copy(stage_v.at[pl.ds(0, SLAB_STEP)],
                    sub_sh.at[pl.ds(sid * SLAB_STEP, SLAB_STEP)])

    @pl.when(sid == NS - 1)
    def _():
        pltpu.sync_copy(
            stage_v.at[pl.ds(SLAB_STEP, LANES)],
            sub_sh.at[pl.ds(NS * SLAB_STEP, LANES)])

    plsc.subcore_barrier()
    pltpu.sync_copy(sub_sh, sub_v)
    scope_bounds.__exit__(None, None, None)
    scope_bounds = jax.named_scope("sc_coarse")
    scope_bounds.__enter__()

    r_lo = [jnp.zeros((LANES,), jnp.int32) for _ in range(QG)]
    r_hi = [jnp.full((LANES,), NSUB, jnp.int32) for _ in range(QG)]
    for _ in range(ITERS_A):
        for q in range(QG):
            active = r_lo[q] < r_hi[q]
            mid = r_lo[q] + ((r_hi[q] - r_lo[q]) >> 1)
            sval = plsc.load_gather(sub_v, [jnp.minimum(mid, NSUB - 1)])
            go = active & (sval < tgts[q])
            r_lo[q] = jnp.where(go, mid + 1, r_lo[q])
            r_hi[q] = jnp.where(active & jnp.logical_not(go), mid, r_hi[q])

    scope_bounds.__exit__(None, None, None)
    scope_bounds = jax.named_scope("sc_win")
    scope_bounds.__enter__()
    for q in range(QG):
        rowi_v[pl.ds(q * LANES, LANES)] = jnp.maximum(r_lo[q] - 1, 0)
    pltpu.async_copy(batch2_hbm.at[rowi_v], win_v, sem0).wait()
    scope_bounds.__exit__(None, None, None)
    scope_bounds = jax.named_scope("sc_fine")
    scope_bounds.__enter__()

    for q in range(QG):
        r_star = r_lo[q]
        at0 = r_star == 0
        lo = jnp.where(at0, 0, 1)
        hi = jnp.where(at0, 0, SUB)
        qrow = q * LANES + lane
        for _ in range(ITERS_B):
            active = lo < hi
            mid = lo + ((hi - lo) >> 1)
            wval = plsc.load_gather(win_v, [qrow, jnp.minimum(mid, SUB - 1)])
            go = active & (wval < tgts[q])
            lo = jnp.where(go, mid + 1, lo)
            hi = jnp.where(active & jnp.logical_not(go), mid, hi)
        bnd_v[pl.ds(q * LANES, LANES)] = jnp.where(
            at0, 0, (r_star - 1) * SUB + lo)

    scope_bounds.__exit__(None, None, None)
    scope_pool = jax.named_scope("sc_pool")
    scope_pool.__enter__()
    rs_w = _sread(bnd_v, 0)
    re_w = _sread(bnd_v, SEG_PER_W)
    nch = (re_w - rs_w + (CHUNK - 1)) // CHUNK

    bufs = (xbuf0, xbuf1)
    sems = (sem0, sem1)

    def issue(k, buf, sem):
        c0 = rs_w + k * CHUNK
        base = jnp.minimum(c0, N - CHUNK)
        pltpu.async_copy(x_hbm.at[pl.ds(base * D, CHUNK * D)], buf, sem)

    @pl.when(nch >= 1)
    def _():
        issue(0, xbuf0, sem0)

    @pl.when(nch >= 2)
    def _():
        issue(1, xbuf1, sem1)

    zeros = tuple(jnp.zeros((LANES,), jnp.float32) for _ in range(DG))
    ninf = tuple(jnp.full((LANES,), -jnp.inf, jnp.float32) for _ in range(DG))

    def flush(seg, sums, maxs):
        for t in range(DG):
            outs_v[pl.ds(seg * D + t * LANES, LANES)] = sums[t]
            outm_v[pl.ds(seg * D + t * LANES, LANES)] = maxs[t]

    def rows(lo, hi, base, buf, sums, maxs):
        def body(off, c):
            cs, cm = c
            ns, nm = [], []
            for t in range(DG):
                xv = buf[pl.ds(off + t * LANES, LANES)]
                ns.append(cs[t] + xv)
                nm.append(jnp.maximum(cm[t], xv))
            return (tuple(ns), tuple(nm))
        return plsc.parallel_loop((lo - base) * D, (hi - base) * D, D,
                                  carry=(sums, maxs), unroll=8)(body)

    def process_chunk(k, buf, sem, carry):
        seg, sums, maxs = carry
        pltpu.make_async_copy(x_hbm.at[pl.ds(0, CHUNK * D)], buf, sem).wait()

        @pl.when(k + 2 < nch)
        def _():
            issue(k + 2, buf, sem)

        c0 = rs_w + k * CHUNK
        base = jnp.minimum(c0, N - CHUNK)
        c1 = jnp.minimum(c0 + CHUNK, re_w)

        def wcond(st):
            wseg = st[0]
            return (wseg < SEG_PER_W) & (_sread(bnd_v, wseg + 1) <= c1)

        def wbody(st):
            wseg, cur, ws, wm = st
            e = _sread(bnd_v, wseg + 1)
            ws, wm = rows(cur, e, base, buf, ws, wm)
            flush(wseg, ws, wm)
            return (wseg + 1, e, zeros, ninf)

        seg, cur, sums, maxs = lax.while_loop(
            wcond, wbody, (seg, c0, sums, maxs))
        sums, maxs = rows(cur, c1, base, buf, sums, maxs)
        return (seg, sums, maxs)

    def pair_body(kk, carry):
        for b in range(2):
            k = 2 * kk + b
            carry = lax.cond(
                k < nch,
                lambda c, k=k, b=b: process_chunk(k, bufs[b], sems[b], c),
                lambda c: c,
                carry)
        return carry

    carry0 = (jnp.int32(0), zeros, ninf)
    seg, _, _ = lax.fori_loop(0, (nch + 1) // 2, pair_body, carry0)

    def tail_cond(st):
        return st[0] < SEG_PER_W

    def tail_body(st):
        flush(st[0], zeros, ninf)
        return (st[0] + 1,)

    lax.while_loop(tail_cond, tail_body, (seg,))

    pltpu.sync_copy(outs_v, sum_hbm.at[pl.ds(seg0 * D, SEG_PER_W * D)])
    pltpu.sync_copy(outm_v, max_hbm.at[pl.ds(seg0 * D, SEG_PER_W * D)])
    scope_pool.__exit__(None, None, None)


def _mlp_body(alpha_ref, ps_ref, pm_ref, w1, b1, w2, b2, w3, b3, w4, b4,
              out_ref):
    a = jax.nn.sigmoid(alpha_ref[0, 0])
    pooled = a * ps_ref[...] + (1.0 - a) * pm_ref[...]
    z = jnp.dot(pooled, w1[...], preferred_element_type=jnp.float32) + b1[...]
    h = jnp.where(z > 0, z, jnp.exp(z) - 1.0)
    z = jnp.dot(h, w2[...], preferred_element_type=jnp.float32) + b2[...]
    h = jnp.where(z > 0, z, jnp.exp(z) - 1.0)
    z = jnp.dot(h, w3[...], preferred_element_type=jnp.float32) + b3[...]
    h = jnp.where(z > 0, z, jnp.exp(z) - 1.0)
    out_ref[...] = (
        jnp.dot(h, w4[...], preferred_element_type=jnp.float32) + b4[...])


def kernel(x, batch, alpha, W1, b1, W2, b2, W3, b3, W4, b4):
    mesh = plsc.VectorSubcoreMesh(core_axis_name="c", subcore_axis_name="s")
    cp = pltpu.CompilerParams()
    if "needs_layout_passes" in pltpu.CompilerParams.__dataclass_fields__:
        cp = dataclasses.replace(cp, needs_layout_passes=False)
    pool = pl.kernel(
        _pool_body,
        out_type=[jax.ShapeDtypeStruct((B * D,), jnp.float32),
                  jax.ShapeDtypeStruct((B * D,), jnp.float32)],
        mesh=mesh,
        compiler_params=cp,
        scratch_types=[
            pltpu.VMEM((BND_WIN,), jnp.int32),
            pltpu.VMEM((SLAB_ROWS, SUB), jnp.int32),
            pltpu.VMEM((4 * LANES,), jnp.int32),
            pltpu.VMEM((NSUB_PAD,), jnp.int32),
            pltpu.VMEM_SHARED((NSUB_PAD,), jnp.int32),
            pltpu.VMEM((BND_WIN,), jnp.int32),
            pltpu.VMEM((BND_WIN, SUB), jnp.int32),
            pltpu.VMEM((CHUNK * D,), jnp.float32),
            pltpu.VMEM((CHUNK * D,), jnp.float32),
            pltpu.VMEM((SEG_PER_W * D,), jnp.float32),
            pltpu.VMEM((SEG_PER_W * D,), jnp.float32),
            pltpu.SemaphoreType.DMA,
            pltpu.SemaphoreType.DMA,
        ],
    )
    batch_pad = jnp.concatenate(
        [batch, jnp.full((N_PAD - N,), PAD_ID, jnp.int32)])
    ps, pm = pool(x.reshape(N * D), batch_pad, batch_pad.reshape(NSUB, SUB))

    out = pl.pallas_call(
        _mlp_body,
        out_shape=jax.ShapeDtypeStruct((B, D), jnp.float32),
    )(alpha.reshape(1, 1), ps.reshape(B, D), pm.reshape(B, D),
      W1, b1.reshape(1, H), W2, b2.reshape(1, H), W3, b3.reshape(1, H),
      W4, b4.reshape(1, D))
    return out


# speedup vs baseline: 1.0173x; 1.0173x over previous
import dataclasses

import jax
import jax.numpy as jnp
from jax import lax
from jax.experimental import pallas as pl
from jax.experimental.pallas import tpu as pltpu
from jax.experimental.pallas import tpu_sc as plsc

N = 100000
D = 128
H = 128
B = 1024

NC = 2
NS = 16
NW = NC * NS
SEG_PER_W = B // NW
LANES = 16
DG = D // LANES
CHUNK = 256
BND_WIN = 48
SUB = 128
N_PAD = 100352
PAD_ID = 2047
NSUB = N_PAD // SUB
NSUB_PAD = 896
SLAB_STEP = 48
SLAB_ROWS = 64
ITERS_A = 10
ITERS_B = 7


def _sread(ref, i):
    base = (i // LANES) * LANES
    win = ref[pl.ds(base, LANES)]
    lane = lax.iota(jnp.int32, LANES)
    return jnp.sum(jnp.where(lane == (i - base), win, 0))


def _pool_body(x_hbm, batch_hbm, batch2_hbm, sum_hbm, max_hbm,
               bnd_v, slab_v, stage_v, sub_v, sub_sh, rowi_v, win_v,
               xbuf0, xbuf1, outs_v, outm_v, sem0, sem1):
    sid = lax.axis_index("s")
    wid = lax.axis_index("c") * NS + sid
    seg0 = wid * SEG_PER_W

    lane = lax.iota(jnp.int32, LANES)
    QG = BND_WIN // LANES
    tgts = [seg0 + q * LANES + lane for q in range(QG)]

    pltpu.sync_copy(batch2_hbm.at[pl.ds(sid * SLAB_STEP, SLAB_ROWS)], slab_v)
    for g in range(4):
        ridx = g * LANES + lane
        stage_v[pl.ds(g * LANES, LANES)] = plsc.load_gather(
            slab_v, [ridx, jnp.zeros((LANES,), jnp.int32)])
    pltpu.sync_copy(stage_v.at[pl.ds(0, SLAB_STEP)],
                    sub_sh.at[pl.ds(sid * SLAB_STEP, SLAB_STEP)])

    @pl.when(sid == NS - 1)
    def _():
        pltpu.sync_copy(
            stage_v.at[pl.ds(SLAB_STEP, LANES)],
            sub_sh.at[pl.ds(NS * SLAB_STEP, LANES)])

    plsc.subcore_barrier()
    pltpu.sync_copy(sub_sh, sub_v)

    r_lo = [jnp.zeros((LANES,), jnp.int32) for _ in range(QG)]
    r_hi = [jnp.full((LANES,), NSUB, jnp.int32) for _ in range(QG)]
    for _ in range(ITERS_A):
        for q in range(QG):
            active = r_lo[q] < r_hi[q]
            mid = r_lo[q] + ((r_hi[q] - r_lo[q]) >> 1)
            sval = plsc.load_gather(sub_v, [jnp.minimum(mid, NSUB - 1)])
            go = active & (sval < tgts[q])
            r_lo[q] = jnp.where(go, mid + 1, r_lo[q])
            r_hi[q] = jnp.where(active & jnp.logical_not(go), mid, r_hi[q])

    for q in range(QG):
        rowi_v[pl.ds(q * LANES, LANES)] = jnp.maximum(r_lo[q] - 1, 0)
    pltpu.async_copy(batch2_hbm.at[rowi_v], win_v, sem0).wait()

    for q in range(QG):
        r_star = r_lo[q]
        at0 = r_star == 0
        lo = jnp.where(at0, 0, 1)
        hi = jnp.where(at0, 0, SUB)
        qrow = q * LANES + lane
        for _ in range(ITERS_B):
            active = lo < hi
            mid = lo + ((hi - lo) >> 1)
            wval = plsc.load_gather(win_v, [qrow, jnp.minimum(mid, SUB - 1)])
            go = active & (wval < tgts[q])
            lo = jnp.where(go, mid + 1, lo)
            hi = jnp.where(active & jnp.logical_not(go), mid, hi)
        bnd_v[pl.ds(q * LANES, LANES)] = jnp.where(
            at0, 0, (r_star - 1) * SUB + lo)

    rs_w = _sread(bnd_v, 0)
    re_w = _sread(bnd_v, SEG_PER_W)
    nch = (re_w - rs_w + (CHUNK - 1)) // CHUNK

    bufs = (xbuf0, xbuf1)
    sems = (sem0, sem1)

    def issue(k, buf, sem):
        c0 = rs_w + k * CHUNK
        base = jnp.minimum(c0, N - CHUNK)
        pltpu.async_copy(x_hbm.at[pl.ds(base * D, CHUNK * D)], buf, sem)

    @pl.when(nch >= 1)
    def _():
        issue(0, xbuf0, sem0)

    @pl.when(nch >= 2)
    def _():
        issue(1, xbuf1, sem1)

    zeros = tuple(jnp.zeros((LANES,), jnp.float32) for _ in range(DG))
    ninf = tuple(jnp.full((LANES,), -jnp.inf, jnp.float32) for _ in range(DG))

    def flush(seg, sums, maxs):
        for t in range(DG):
            outs_v[pl.ds(seg * D + t * LANES, LANES)] = sums[t]
            outm_v[pl.ds(seg * D + t * LANES, LANES)] = maxs[t]

    def rows(lo, hi, base, buf, sums, maxs):
        def body(r, c):
            cs, cm = c
            off = (r - base) * D
            ns, nm = [], []
            for t in range(DG):
                xv = buf[pl.ds(off + t * LANES, LANES)]
                ns.append(cs[t] + xv)
                nm.append(jnp.maximum(cm[t], xv))
            return (tuple(ns), tuple(nm))
        return plsc.parallel_loop(lo, hi, carry=(sums, maxs), unroll=4)(body)

    def process_chunk(k, buf, sem, carry):
        seg, sums, maxs = carry
        pltpu.make_async_copy(x_hbm.at[pl.ds(0, CHUNK * D)], buf, sem).wait()

        @pl.when(k + 2 < nch)
        def _():
            issue(k + 2, buf, sem)

        c0 = rs_w + k * CHUNK
        base = jnp.minimum(c0, N - CHUNK)
        c1 = jnp.minimum(c0 + CHUNK, re_w)

        def wcond(st):
            wseg = st[0]
            return (wseg < SEG_PER_W) & (_sread(bnd_v, wseg + 1) <= c1)

        def wbody(st):
            wseg, cur, ws, wm = st
            e = _sread(bnd_v, wseg + 1)
            ws, wm = rows(cur, e, base, buf, ws, wm)
            flush(wseg, ws, wm)
            return (wseg + 1, e, zeros, ninf)

        seg, cur, sums, maxs = lax.while_loop(
            wcond, wbody, (seg, c0, sums, maxs))
        sums, maxs = rows(cur, c1, base, buf, sums, maxs)
        return (seg, sums, maxs)

    def pair_body(kk, carry):
        for b in range(2):
            k = 2 * kk + b
            carry = lax.cond(
                k < nch,
                lambda c, k=k, b=b: process_chunk(k, bufs[b], sems[b], c),
                lambda c: c,
                carry)
        return carry

    carry0 = (jnp.int32(0), zeros, ninf)
    seg, _, _ = lax.fori_loop(0, (nch + 1) // 2, pair_body, carry0)

    def tail_cond(st):
        return st[0] < SEG_PER_W

    def tail_body(st):
        flush(st[0], zeros, ninf)
        return (st[0] + 1,)

    lax.while_loop(tail_cond, tail_body, (seg,))

    pltpu.sync_copy(outs_v, sum_hbm.at[pl.ds(seg0 * D, SEG_PER_W * D)])
    pltpu.sync_copy(outm_v, max_hbm.at[pl.ds(seg0 * D, SEG_PER_W * D)])


def _mlp_body(alpha_ref, ps_ref, pm_ref, w1, b1, w2, b2, w3, b3, w4, b4,
              out_ref):
    a = jax.nn.sigmoid(alpha_ref[0, 0])
    pooled = a * ps_ref[...] + (1.0 - a) * pm_ref[...]
    z = jnp.dot(pooled, w1[...], preferred_element_type=jnp.float32) + b1[...]
    h = jnp.where(z > 0, z, jnp.exp(z) - 1.0)
    z = jnp.dot(h, w2[...], preferred_element_type=jnp.float32) + b2[...]
    h = jnp.where(z > 0, z, jnp.exp(z) - 1.0)
    z = jnp.dot(h, w3[...], preferred_element_type=jnp.float32) + b3[...]
    h = jnp.where(z > 0, z, jnp.exp(z) - 1.0)
    out_ref[...] = (
        jnp.dot(h, w4[...], preferred_element_type=jnp.float32) + b4[...])


def kernel(x, batch, alpha, W1, b1, W2, b2, W3, b3, W4, b4):
    mesh = plsc.VectorSubcoreMesh(core_axis_name="c", subcore_axis_name="s")
    cp = pltpu.CompilerParams()
    if "needs_layout_passes" in pltpu.CompilerParams.__dataclass_fields__:
        cp = dataclasses.replace(cp, needs_layout_passes=False)
    pool = pl.kernel(
        _pool_body,
        out_type=[jax.ShapeDtypeStruct((B * D,), jnp.float32),
                  jax.ShapeDtypeStruct((B * D,), jnp.float32)],
        mesh=mesh,
        compiler_params=cp,
        scratch_types=[
            pltpu.VMEM((BND_WIN,), jnp.int32),
            pltpu.VMEM((SLAB_ROWS, SUB), jnp.int32),
            pltpu.VMEM((4 * LANES,), jnp.int32),
            pltpu.VMEM((NSUB_PAD,), jnp.int32),
            pltpu.VMEM_SHARED((NSUB_PAD,), jnp.int32),
            pltpu.VMEM((BND_WIN,), jnp.int32),
            pltpu.VMEM((BND_WIN, SUB), jnp.int32),
            pltpu.VMEM((CHUNK * D,), jnp.float32),
            pltpu.VMEM((CHUNK * D,), jnp.float32),
            pltpu.VMEM((SEG_PER_W * D,), jnp.float32),
            pltpu.VMEM((SEG_PER_W * D,), jnp.float32),
            pltpu.SemaphoreType.DMA,
            pltpu.SemaphoreType.DMA,
        ],
    )
    batch_pad = jnp.concatenate(
        [batch, jnp.full((N_PAD - N,), PAD_ID, jnp.int32)])
    ps, pm = pool(x.reshape(N * D), batch_pad, batch_pad.reshape(NSUB, SUB))

    out = pl.pallas_call(
        _mlp_body,
        out_shape=jax.ShapeDtypeStruct((B, D), jnp.float32),
    )(alpha.reshape(1, 1), ps.reshape(B, D), pm.reshape(B, D),
      W1, b1.reshape(1, H), W2, b2.reshape(1, H), W3, b3.reshape(1, H),
      W4, b4.reshape(1, D))
    return out
